# baseline (speedup 1.0000x reference)
import dataclasses
import functools

import jax
import jax.numpy as jnp
from jax import lax
from jax.experimental import pallas as pl
from jax.experimental.pallas import tpu as pltpu
from jax.experimental.pallas import tpu_sc as plsc

NC = 2
NS = 16
NW = NC * NS
CH = 128
LANES = 16

_vector_mesh = plsc.VectorSubcoreMesh(
    core_axis_name="core", subcore_axis_name="subcore")

_sc_params = pltpu.CompilerParams(
    needs_layout_passes=False, use_tc_tiling_on_sc=False)


def _hist_sc(dst1d, npad, tile_e):
    @functools.partial(
        pl.kernel,
        out_type=jax.ShapeDtypeStruct((NW, npad), jnp.float32),
        mesh=_vector_mesh,
        compiler_params=_sc_params,
        scratch_types=[
            pltpu.VMEM((tile_e,), jnp.int32),
            pltpu.VMEM((npad,), jnp.float32),
        ],
    )
    def hist_k(dst_hbm, out_hbm, idx_v, hist_v):
        c = lax.axis_index("core")
        s = lax.axis_index("subcore")
        w = c * NS + s
        pltpu.sync_copy(dst_hbm.at[pl.ds(w * tile_e, tile_e)], idx_v)

        @pl.loop(0, npad, step=LANES)
        def _(i):
            hist_v[pl.ds(i, LANES)] = jnp.zeros((LANES,), jnp.float32)

        ones = jnp.ones((LANES,), jnp.float32)

        @pl.loop(0, tile_e, step=LANES)
        def _(e):
            idx = idx_v[pl.ds(e, LANES)]
            plsc.addupdate_scatter(hist_v, [idx], ones)

        pltpu.sync_copy(hist_v, out_hbm.at[w])

    return hist_k(dst1d)


def _prop_sc(g, src2d, dst2d, npad, kr, rpt):
    hd = g.shape[1]

    @functools.partial(
        pl.kernel,
        out_type=jax.ShapeDtypeStruct((NC, npad, hd), jnp.float32),
        mesh=_vector_mesh,
        compiler_params=_sc_params,
        scratch_types=[
            pltpu.VMEM((kr, CH), jnp.int32),
            pltpu.VMEM((kr, CH), jnp.int32),
            pltpu.VMEM((CH, hd), jnp.float32),
            pltpu.VMEM((CH, hd), jnp.float32),
            pltpu.VMEM((rpt, hd), jnp.float32),
            pltpu.VMEM_SHARED((npad, hd), jnp.float32),
            pltpu.SemaphoreType.DMA,
            pltpu.SemaphoreType.DMA,
        ],
    )
    def prop_k(g_hbm, src_hbm, dst_hbm, out_hbm,
               srcv, dstv, buf0, buf1, stage, accum, sem0, sem1):
        c = lax.axis_index("core")
        s = lax.axis_index("subcore")
        w = c * NS + s

        cp_s = pltpu.async_copy(src_hbm.at[pl.ds(w * kr, kr)], srcv, sem0)
        cp_d = pltpu.async_copy(dst_hbm.at[pl.ds(w * kr, kr)], dstv, sem1)

        @pl.loop(0, rpt)
        def _(i):
            stage[i, :] = jnp.zeros((hd,), jnp.float32)

        pltpu.sync_copy(stage, accum.at[pl.ds(s * rpt, rpt)])
        cp_s.wait()
        cp_d.wait()
        plsc.subcore_barrier()

        pltpu.async_copy(g_hbm.at[srcv.at[0]], buf0, sem0)

        @pl.loop(0, kr, step=2)
        def _(j):
            pltpu.async_copy(g_hbm.at[srcv.at[j + 1]], buf1, sem1)
            pltpu.make_async_copy(g_hbm.at[srcv.at[0]], buf0, sem0).wait()
            pltpu.sync_copy(buf0, accum.at[dstv.at[j]], add=True)

            @pl.when(j + 2 < kr)
            def _():
                pltpu.async_copy(g_hbm.at[srcv.at[j + 2]], buf0, sem0)

            pltpu.make_async_copy(g_hbm.at[srcv.at[0]], buf1, sem1).wait()
            pltpu.sync_copy(buf1, accum.at[dstv.at[j + 1]], add=True)

        plsc.subcore_barrier()
        pltpu.sync_copy(accum.at[pl.ds(s * rpt, rpt)], stage)
        pltpu.sync_copy(stage, out_hbm.at[c, pl.ds(s * rpt, rpt)])

    return prop_k(g, src2d, dst2d)


def _tc_mm1(x, w1, bn):
    n, f = x.shape
    hd = w1.shape[1]

    def body(x_ref, w_ref, h_ref):
        h_ref[...] = jnp.dot(x_ref[...], w_ref[...],
                             preferred_element_type=jnp.float32)

    return pl.pallas_call(
        body,
        grid=(n // bn,),
        in_specs=[
            pl.BlockSpec((bn, f), lambda i: (i, 0)),
            pl.BlockSpec((f, hd), lambda i: (0, 0)),
        ],
        out_specs=pl.BlockSpec((bn, hd), lambda i: (i, 0)),
        out_shape=jax.ShapeDtypeStruct((n, hd), jnp.float32),
    )(x, w1)


def _tc_dis_scale(hist, h1):
    nw, npad = hist.shape
    n, hd = h1.shape

    def body(hist_ref, h_ref, dis_ref, g_ref):
        deg = jnp.sum(hist_ref[...], axis=0) + 1.0
        dis = lax.rsqrt(deg)
        disb = jnp.broadcast_to(dis[:, None], (npad, hd))
        dis_ref[...] = disb
        g_ref[...] = h_ref[...] * disb[:n, :]

    return pl.pallas_call(
        body,
        in_specs=[
            pl.BlockSpec((nw, npad), lambda: (0, 0)),
            pl.BlockSpec((n, hd), lambda: (0, 0)),
        ],
        out_specs=[
            pl.BlockSpec((npad, hd), lambda: (0, 0)),
            pl.BlockSpec((n, hd), lambda: (0, 0)),
        ],
        out_shape=[
            jax.ShapeDtypeStruct((npad, hd), jnp.float32),
            jax.ShapeDtypeStruct((n, hd), jnp.float32),
        ],
    )(hist, h1)


def _tc2(p, g, dis, w2, b1, bn):
    n, hd = g.shape
    grid = n // bn

    def body(p_ref, g_ref, dis_ref, w_ref, b_ref, o_ref):
        s = p_ref[0] + p_ref[1] + g_ref[...]
        a = jnp.maximum(dis_ref[...] * s + b_ref[...], 0.0)
        h = jnp.dot(a, w_ref[...], preferred_element_type=jnp.float32)
        o_ref[...] = h * dis_ref[...]

    return pl.pallas_call(
        body,
        grid=(grid,),
        in_specs=[
            pl.BlockSpec((NC, bn, hd), lambda i: (0, i, 0)),
            pl.BlockSpec((bn, hd), lambda i: (i, 0)),
            pl.BlockSpec((bn, hd), lambda i: (i, 0)),
            pl.BlockSpec((hd, hd), lambda i: (0, 0)),
            pl.BlockSpec((1, hd), lambda i: (0, 0)),
        ],
        out_specs=pl.BlockSpec((bn, hd), lambda i: (i, 0)),
        out_shape=jax.ShapeDtypeStruct((n, hd), jnp.float32),
    )(p, g, dis, w2, b1)


def _tc3(q, g2, dis, b2, wl, bl, bn):
    n, hd = g2.shape
    co = wl.shape[1]
    grid = n // bn

    def body(q_ref, g_ref, dis_ref, b_ref, w_ref, bl_ref, o_ref):
        s = q_ref[0] + q_ref[1] + g_ref[...]
        a = jnp.maximum(dis_ref[...] * s + b_ref[...], 0.0)
        logits = jnp.dot(a, w_ref[...], preferred_element_type=jnp.float32)
        logits = logits + bl_ref[...]
        m = jnp.max(logits, axis=1, keepdims=True)
        lse = m + jnp.log(jnp.sum(jnp.exp(logits - m), axis=1, keepdims=True))
        o_ref[...] = logits - lse

    return pl.pallas_call(
        body,
        grid=(grid,),
        in_specs=[
            pl.BlockSpec((NC, bn, hd), lambda i: (0, i, 0)),
            pl.BlockSpec((bn, hd), lambda i: (i, 0)),
            pl.BlockSpec((bn, hd), lambda i: (i, 0)),
            pl.BlockSpec((1, hd), lambda i: (0, 0)),
            pl.BlockSpec((hd, co), lambda i: (0, 0)),
            pl.BlockSpec((1, co), lambda i: (0, 0)),
        ],
        out_specs=pl.BlockSpec((bn, co), lambda i: (i, 0)),
        out_shape=jax.ShapeDtypeStruct((n, co), jnp.float32),
    )(q, g2, dis, b2, wl, bl)


def kernel(x, edge_index, W1, b1, W2, b2, Wl, bl):
    n = x.shape[0]
    e = edge_index.shape[1]

    tile_e = -(-e // NW)
    tile_e = -(-tile_e // (2 * CH)) * (2 * CH)
    ep = tile_e * NW
    kr = tile_e // CH
    npad = -(-(n + 1) // (8 * NS)) * (8 * NS)
    rpt = npad // NS

    src = edge_index[0]
    dst = edge_index[1]
    e1 = -(-e // NW) * NW
    if e1 != e:
        src = jnp.concatenate([src, jnp.zeros((e1 - e,), jnp.int32)])
        dst = jnp.concatenate([dst, jnp.full((e1 - e,), n, jnp.int32)])
    ept = e1 // NW
    padt = tile_e - ept
    dummy_dst = n + jnp.arange(padt, dtype=jnp.int32) % (npad - n)
    src2d = jnp.concatenate(
        [src.reshape(NW, ept), jnp.zeros((NW, padt), jnp.int32)], axis=1)
    dst2d = jnp.concatenate(
        [dst.reshape(NW, ept), jnp.broadcast_to(dummy_dst, (NW, padt))], axis=1)
    src2d = src2d.reshape(NW * kr, CH)
    dst2d = dst2d.reshape(NW * kr, CH)

    hist = _hist_sc(dst2d.reshape(ep), npad, tile_e)

    bn = 1000 if n % 1000 == 0 else 8
    h1 = _tc_mm1(x, W1, bn)
    dis, g1 = _tc_dis_scale(hist, h1)
    p = _prop_sc(g1, src2d, dst2d, npad, kr, rpt)
    g2 = _tc2(p, g1, dis, W2, b1.reshape(1, -1), bn)
    q = _prop_sc(g2, src2d, dst2d, npad, kr, rpt)
    return _tc3(q, g2, dis, b2.reshape(1, -1), Wl, bl.reshape(1, -1), bn)


# speedup vs baseline: 1.0150x; 1.0150x over previous
import dataclasses
import functools

import jax
import jax.numpy as jnp
from jax import lax
from jax.experimental import pallas as pl
from jax.experimental.pallas import tpu as pltpu
from jax.experimental.pallas import tpu_sc as plsc

NC = 2
NS = 16
NW = NC * NS
CH = 128
LANES = 16

_vector_mesh = plsc.VectorSubcoreMesh(
    core_axis_name="core", subcore_axis_name="subcore")

_sc_params = pltpu.CompilerParams(
    needs_layout_passes=False, use_tc_tiling_on_sc=False)


def _hist_sc(dst1d, npad, tile_e):
    @functools.partial(
        pl.kernel,
        out_type=jax.ShapeDtypeStruct((NW, npad), jnp.float32),
        mesh=_vector_mesh,
        compiler_params=_sc_params,
        scratch_types=[
            pltpu.VMEM((tile_e,), jnp.int32),
            pltpu.VMEM((npad,), jnp.float32),
        ],
    )
    def hist_k(dst_hbm, out_hbm, idx_v, hist_v):
        c = lax.axis_index("core")
        s = lax.axis_index("subcore")
        w = c * NS + s
        pltpu.sync_copy(dst_hbm.at[pl.ds(w * tile_e, tile_e)], idx_v)

        @pl.loop(0, npad, step=LANES)
        def _(i):
            hist_v[pl.ds(i, LANES)] = jnp.zeros((LANES,), jnp.float32)

        ones = jnp.ones((LANES,), jnp.float32)

        @pl.loop(0, tile_e, step=LANES)
        def _(e):
            idx = idx_v[pl.ds(e, LANES)]
            plsc.addupdate_scatter(hist_v, [idx], ones)

        pltpu.sync_copy(hist_v, out_hbm.at[w])

    return hist_k(dst1d)


def _prop_sc(g, src2d, dst2d, npad, kr, rpt):
    hd = g.shape[1]

    @functools.partial(
        pl.kernel,
        out_type=jax.ShapeDtypeStruct((NC, npad, hd), jnp.float32),
        mesh=_vector_mesh,
        compiler_params=_sc_params,
        scratch_types=[
            pltpu.VMEM((kr, CH), jnp.int32),
            pltpu.VMEM((kr, CH), jnp.int32),
            pltpu.VMEM((CH, hd), jnp.float32),
            pltpu.VMEM((CH, hd), jnp.float32),
            pltpu.VMEM((rpt, hd), jnp.float32),
            pltpu.VMEM_SHARED((npad, hd), jnp.float32),
            pltpu.SemaphoreType.DMA,
            pltpu.SemaphoreType.DMA,
        ],
    )
    def prop_k(g_hbm, src_hbm, dst_hbm, out_hbm,
               srcv, dstv, buf0, buf1, stage, accum, sem0, sem1):
        c = lax.axis_index("core")
        s = lax.axis_index("subcore")
        w = c * NS + s

        cp_s = pltpu.async_copy(src_hbm.at[pl.ds(w * kr, kr)], srcv, sem0)
        cp_d = pltpu.async_copy(dst_hbm.at[pl.ds(w * kr, kr)], dstv, sem1)

        @pl.loop(0, rpt)
        def _(i):
            stage[i, :] = jnp.zeros((hd,), jnp.float32)

        pltpu.sync_copy(stage, accum.at[pl.ds(s * rpt, rpt)])
        cp_s.wait()
        cp_d.wait()
        plsc.subcore_barrier()

        pltpu.async_copy(g_hbm.at[srcv.at[0]], buf0, sem0)

        @pl.loop(0, kr, step=2)
        def _(j):
            pltpu.async_copy(g_hbm.at[srcv.at[j + 1]], buf1, sem1)
            pltpu.make_async_copy(g_hbm.at[srcv.at[0]], buf0, sem0).wait()
            pltpu.sync_copy(buf0, accum.at[dstv.at[j]], add=True)

            @pl.when(j + 2 < kr)
            def _():
                pltpu.async_copy(g_hbm.at[srcv.at[j + 2]], buf0, sem0)

            pltpu.make_async_copy(g_hbm.at[srcv.at[0]], buf1, sem1).wait()
            pltpu.sync_copy(buf1, accum.at[dstv.at[j + 1]], add=True)

        plsc.subcore_barrier()
        pltpu.sync_copy(accum.at[pl.ds(s * rpt, rpt)], stage)
        pltpu.sync_copy(stage, out_hbm.at[c, pl.ds(s * rpt, rpt)])

    return prop_k(g, src2d, dst2d)


def _tc_front(x, w1, hist):
    nw, npad = hist.shape
    n, f = x.shape
    hd = w1.shape[1]

    def body(x_ref, w_ref, hist_ref, dis_ref, g_ref):
        deg = jnp.sum(hist_ref[...], axis=0) + 1.0
        dis = lax.rsqrt(deg)
        disb = jnp.broadcast_to(dis[:, None], (npad, hd))
        dis_ref[...] = disb
        h = jnp.dot(x_ref[...], w_ref[...], preferred_element_type=jnp.float32)
        g_ref[...] = h * disb[:n, :]

    return pl.pallas_call(
        body,
        in_specs=[
            pl.BlockSpec((n, f), lambda: (0, 0)),
            pl.BlockSpec((f, hd), lambda: (0, 0)),
            pl.BlockSpec((nw, npad), lambda: (0, 0)),
        ],
        out_specs=[
            pl.BlockSpec((npad, hd), lambda: (0, 0)),
            pl.BlockSpec((n, hd), lambda: (0, 0)),
        ],
        out_shape=[
            jax.ShapeDtypeStruct((npad, hd), jnp.float32),
            jax.ShapeDtypeStruct((n, hd), jnp.float32),
        ],
    )(x, w1, hist)


def _tc2(p, g, dis, w2, b1, bn):
    n, hd = g.shape
    grid = n // bn

    def body(p_ref, g_ref, dis_ref, w_ref, b_ref, o_ref):
        s = p_ref[0] + p_ref[1] + g_ref[...]
        a = jnp.maximum(dis_ref[...] * s + b_ref[...], 0.0)
        h = jnp.dot(a, w_ref[...], preferred_element_type=jnp.float32)
        o_ref[...] = h * dis_ref[...]

    return pl.pallas_call(
        body,
        grid=(grid,),
        in_specs=[
            pl.BlockSpec((NC, bn, hd), lambda i: (0, i, 0)),
            pl.BlockSpec((bn, hd), lambda i: (i, 0)),
            pl.BlockSpec((bn, hd), lambda i: (i, 0)),
            pl.BlockSpec((hd, hd), lambda i: (0, 0)),
            pl.BlockSpec((1, hd), lambda i: (0, 0)),
        ],
        out_specs=pl.BlockSpec((bn, hd), lambda i: (i, 0)),
        out_shape=jax.ShapeDtypeStruct((n, hd), jnp.float32),
    )(p, g, dis, w2, b1)


def _tc3(q, g2, dis, b2, wl, bl, bn):
    n, hd = g2.shape
    co = wl.shape[1]
    grid = n // bn

    def body(q_ref, g_ref, dis_ref, b_ref, w_ref, bl_ref, o_ref):
        s = q_ref[0] + q_ref[1] + g_ref[...]
        a = jnp.maximum(dis_ref[...] * s + b_ref[...], 0.0)
        logits = jnp.dot(a, w_ref[...], preferred_element_type=jnp.float32)
        logits = logits + bl_ref[...]
        m = jnp.max(logits, axis=1, keepdims=True)
        lse = m + jnp.log(jnp.sum(jnp.exp(logits - m), axis=1, keepdims=True))
        o_ref[...] = logits - lse

    return pl.pallas_call(
        body,
        grid=(grid,),
        in_specs=[
            pl.BlockSpec((NC, bn, hd), lambda i: (0, i, 0)),
            pl.BlockSpec((bn, hd), lambda i: (i, 0)),
            pl.BlockSpec((bn, hd), lambda i: (i, 0)),
            pl.BlockSpec((1, hd), lambda i: (0, 0)),
            pl.BlockSpec((hd, co), lambda i: (0, 0)),
            pl.BlockSpec((1, co), lambda i: (0, 0)),
        ],
        out_specs=pl.BlockSpec((bn, co), lambda i: (i, 0)),
        out_shape=jax.ShapeDtypeStruct((n, co), jnp.float32),
    )(q, g2, dis, b2, wl, bl)


def kernel(x, edge_index, W1, b1, W2, b2, Wl, bl):
    n = x.shape[0]
    e = edge_index.shape[1]

    tile_e = -(-e // NW)
    tile_e = -(-tile_e // (2 * CH)) * (2 * CH)
    ep = tile_e * NW
    kr = tile_e // CH
    npad = -(-(n + 1) // (8 * NS)) * (8 * NS)
    rpt = npad // NS

    src = edge_index[0]
    dst = edge_index[1]
    e1 = -(-e // NW) * NW
    if e1 != e:
        src = jnp.concatenate([src, jnp.zeros((e1 - e,), jnp.int32)])
        dst = jnp.concatenate([dst, jnp.full((e1 - e,), n, jnp.int32)])
    ept = e1 // NW
    padt = tile_e - ept
    dummy_dst = n + jnp.arange(padt, dtype=jnp.int32) % (npad - n)
    src2d = jnp.concatenate(
        [src.reshape(NW, ept), jnp.zeros((NW, padt), jnp.int32)], axis=1)
    dst2d = jnp.concatenate(
        [dst.reshape(NW, ept), jnp.broadcast_to(dummy_dst, (NW, padt))], axis=1)
    src2d = src2d.reshape(NW * kr, CH)
    dst2d = dst2d.reshape(NW * kr, CH)

    hist = _hist_sc(dst2d.reshape(ep), npad, tile_e)

    bn = 1000 if n % 1000 == 0 else 8
    dis, g1 = _tc_front(x, W1, hist)
    p = _prop_sc(g1, src2d, dst2d, npad, kr, rpt)
    g2 = _tc2(p, g1, dis, W2, b1.reshape(1, -1), bn)
    q = _prop_sc(g2, src2d, dst2d, npad, kr, rpt)
    return _tc3(q, g2, dis, b2.reshape(1, -1), Wl, bl.reshape(1, -1), bn)


# speedup vs baseline: 1.1520x; 1.1350x over previous
import dataclasses
import functools

import jax
import jax.numpy as jnp
from jax import lax
from jax.experimental import pallas as pl
from jax.experimental.pallas import tpu as pltpu
from jax.experimental.pallas import tpu_sc as plsc

NC = 2
NS = 16
NW = NC * NS
CH = 128
LANES = 16

_vector_mesh = plsc.VectorSubcoreMesh(
    core_axis_name="core", subcore_axis_name="subcore")

_sc_params = pltpu.CompilerParams(
    needs_layout_passes=False, use_tc_tiling_on_sc=False)


def _hist_sc(dst1d, npad, tile_e):
    @functools.partial(
        pl.kernel,
        out_type=jax.ShapeDtypeStruct((NW, npad), jnp.float32),
        mesh=_vector_mesh,
        compiler_params=_sc_params,
        scratch_types=[
            pltpu.VMEM((tile_e,), jnp.int32),
            pltpu.VMEM((npad,), jnp.float32),
        ],
    )
    def hist_k(dst_hbm, out_hbm, idx_v, hist_v):
        c = lax.axis_index("core")
        s = lax.axis_index("subcore")
        w = c * NS + s
        pltpu.sync_copy(dst_hbm.at[pl.ds(w * tile_e, tile_e)], idx_v)

        @pl.loop(0, npad, step=LANES)
        def _(i):
            hist_v[pl.ds(i, LANES)] = jnp.zeros((LANES,), jnp.float32)

        ones = jnp.ones((LANES,), jnp.float32)

        @pl.loop(0, tile_e, step=LANES)
        def _(e):
            idx = idx_v[pl.ds(e, LANES)]
            plsc.addupdate_scatter(hist_v, [idx], ones)

        pltpu.sync_copy(hist_v, out_hbm.at[w])

    return hist_k(dst1d)


def _prop_sc(g, src2d, dst2d, npad, kr, rpt):
    hd = g.shape[1]

    @functools.partial(
        pl.kernel,
        out_type=jax.ShapeDtypeStruct((NC, npad, hd), jnp.float32),
        mesh=_vector_mesh,
        compiler_params=_sc_params,
        scratch_types=[
            pltpu.VMEM((kr, CH), jnp.int32),
            pltpu.VMEM((kr, CH), jnp.int32),
            pltpu.VMEM((4, CH, hd), jnp.float32),
            pltpu.VMEM((rpt, hd), jnp.float32),
            pltpu.VMEM_SHARED((npad, hd), jnp.float32),
            pltpu.SemaphoreType.DMA((4,)),
            pltpu.SemaphoreType.DMA((4,)),
        ],
    )
    def prop_k(g_hbm, src_hbm, dst_hbm, out_hbm,
               srcv, dstv, bufs, stage, accum, gsem, ssem):
        c = lax.axis_index("core")
        s = lax.axis_index("subcore")
        w = c * NS + s

        cp_s = pltpu.async_copy(src_hbm.at[pl.ds(w * kr, kr)], srcv, gsem.at[0])
        cp_d = pltpu.async_copy(dst_hbm.at[pl.ds(w * kr, kr)], dstv, gsem.at[1])

        @pl.loop(0, rpt)
        def _(i):
            stage[i, :] = jnp.zeros((hd,), jnp.float32)

        pltpu.sync_copy(stage, accum.at[pl.ds(s * rpt, rpt)])
        cp_s.wait()
        cp_d.wait()
        plsc.subcore_barrier()

        def gather(m, k):
            pltpu.async_copy(g_hbm.at[srcv.at[m]], bufs.at[k], gsem.at[k])

        def gwait(k):
            pltpu.make_async_copy(g_hbm.at[srcv.at[0]], bufs.at[k],
                                  gsem.at[k]).wait()

        def scat(m, k):
            pltpu.async_copy(bufs.at[k], accum.at[dstv.at[m]], ssem.at[k],
                             add=True)

        def swait(k):
            pltpu.make_async_copy(bufs.at[k], accum.at[dstv.at[0]],
                                  ssem.at[k]).wait()

        gather(0, 0)
        gather(1, 1)

        @pl.loop(0, kr, step=4)
        def _(j):
            for k in range(4):
                m = j + k

                @pl.when(m >= 2)
                def _():
                    swait((k + 2) % 4)

                @pl.when(m + 2 < kr)
                def _():
                    gather(m + 2, (k + 2) % 4)

                gwait(k)
                scat(m, k)

        swait((kr - 2) % 4)
        swait((kr - 1) % 4)
        plsc.subcore_barrier()
        pltpu.sync_copy(accum.at[pl.ds(s * rpt, rpt)], stage)
        pltpu.sync_copy(stage, out_hbm.at[c, pl.ds(s * rpt, rpt)])

    return prop_k(g, src2d, dst2d)


def _tc_front(x, w1, hist):
    nw, npad = hist.shape
    n, f = x.shape
    hd = w1.shape[1]

    def body(x_ref, w_ref, hist_ref, dis_ref, g_ref):
        deg = jnp.sum(hist_ref[...], axis=0) + 1.0
        dis = lax.rsqrt(deg)
        disb = jnp.broadcast_to(dis[:, None], (npad, hd))
        dis_ref[...] = disb
        h = jnp.dot(x_ref[...], w_ref[...], preferred_element_type=jnp.float32)
        g_ref[...] = h * disb[:n, :]

    return pl.pallas_call(
        body,
        in_specs=[
            pl.BlockSpec((n, f), lambda: (0, 0)),
            pl.BlockSpec((f, hd), lambda: (0, 0)),
            pl.BlockSpec((nw, npad), lambda: (0, 0)),
        ],
        out_specs=[
            pl.BlockSpec((npad, hd), lambda: (0, 0)),
            pl.BlockSpec((n, hd), lambda: (0, 0)),
        ],
        out_shape=[
            jax.ShapeDtypeStruct((npad, hd), jnp.float32),
            jax.ShapeDtypeStruct((n, hd), jnp.float32),
        ],
    )(x, w1, hist)


def _tc2(p, g, dis, w2, b1, bn):
    n, hd = g.shape
    grid = n // bn

    def body(p_ref, g_ref, dis_ref, w_ref, b_ref, o_ref):
        s = p_ref[0] + p_ref[1] + g_ref[...]
        a = jnp.maximum(dis_ref[...] * s + b_ref[...], 0.0)
        h = jnp.dot(a, w_ref[...], preferred_element_type=jnp.float32)
        o_ref[...] = h * dis_ref[...]

    return pl.pallas_call(
        body,
        grid=(grid,),
        in_specs=[
            pl.BlockSpec((NC, bn, hd), lambda i: (0, i, 0)),
            pl.BlockSpec((bn, hd), lambda i: (i, 0)),
            pl.BlockSpec((bn, hd), lambda i: (i, 0)),
            pl.BlockSpec((hd, hd), lambda i: (0, 0)),
            pl.BlockSpec((1, hd), lambda i: (0, 0)),
        ],
        out_specs=pl.BlockSpec((bn, hd), lambda i: (i, 0)),
        out_shape=jax.ShapeDtypeStruct((n, hd), jnp.float32),
    )(p, g, dis, w2, b1)


def _tc3(q, g2, dis, b2, wl, bl, bn):
    n, hd = g2.shape
    co = wl.shape[1]
    grid = n // bn

    def body(q_ref, g_ref, dis_ref, b_ref, w_ref, bl_ref, o_ref):
        s = q_ref[0] + q_ref[1] + g_ref[...]
        a = jnp.maximum(dis_ref[...] * s + b_ref[...], 0.0)
        logits = jnp.dot(a, w_ref[...], preferred_element_type=jnp.float32)
        logits = logits + bl_ref[...]
        m = jnp.max(logits, axis=1, keepdims=True)
        lse = m + jnp.log(jnp.sum(jnp.exp(logits - m), axis=1, keepdims=True))
        o_ref[...] = logits - lse

    return pl.pallas_call(
        body,
        grid=(grid,),
        in_specs=[
            pl.BlockSpec((NC, bn, hd), lambda i: (0, i, 0)),
            pl.BlockSpec((bn, hd), lambda i: (i, 0)),
            pl.BlockSpec((bn, hd), lambda i: (i, 0)),
            pl.BlockSpec((1, hd), lambda i: (0, 0)),
            pl.BlockSpec((hd, co), lambda i: (0, 0)),
            pl.BlockSpec((1, co), lambda i: (0, 0)),
        ],
        out_specs=pl.BlockSpec((bn, co), lambda i: (i, 0)),
        out_shape=jax.ShapeDtypeStruct((n, co), jnp.float32),
    )(q, g2, dis, b2, wl, bl)


def kernel(x, edge_index, W1, b1, W2, b2, Wl, bl):
    n = x.shape[0]
    e = edge_index.shape[1]

    tile_e = -(-e // NW)
    tile_e = -(-tile_e // (2 * CH)) * (2 * CH)
    ep = tile_e * NW
    kr = tile_e // CH
    npad = -(-(n + 1) // (8 * NS)) * (8 * NS)
    rpt = npad // NS

    src = edge_index[0]
    dst = edge_index[1]
    e1 = -(-e // NW) * NW
    if e1 != e:
        src = jnp.concatenate([src, jnp.zeros((e1 - e,), jnp.int32)])
        dst = jnp.concatenate([dst, jnp.full((e1 - e,), n, jnp.int32)])
    ept = e1 // NW
    padt = tile_e - ept
    dummy_dst = n + jnp.arange(padt, dtype=jnp.int32) % (npad - n)
    src2d = jnp.concatenate(
        [src.reshape(NW, ept), jnp.zeros((NW, padt), jnp.int32)], axis=1)
    dst2d = jnp.concatenate(
        [dst.reshape(NW, ept), jnp.broadcast_to(dummy_dst, (NW, padt))], axis=1)
    src2d = src2d.reshape(NW * kr, CH)
    dst2d = dst2d.reshape(NW * kr, CH)

    hist = _hist_sc(dst2d.reshape(ep), npad, tile_e)

    bn = 2000 if n % 2000 == 0 else 8
    dis, g1 = _tc_front(x, W1, hist)
    p = _prop_sc(g1, src2d, dst2d, npad, kr, rpt)
    g2 = _tc2(p, g1, dis, W2, b1.reshape(1, -1), bn)
    q = _prop_sc(g2, src2d, dst2d, npad, kr, rpt)
    return _tc3(q, g2, dis, b2.reshape(1, -1), Wl, bl.reshape(1, -1), bn)


# speedup vs baseline: 1.2220x; 1.0608x over previous
import dataclasses
import functools

import jax
import jax.numpy as jnp
from jax import lax
from jax.experimental import pallas as pl
from jax.experimental.pallas import tpu as pltpu
from jax.experimental.pallas import tpu_sc as plsc

NC = 2
NS = 16
NW = NC * NS
CH = 128
LANES = 16

_vector_mesh = plsc.VectorSubcoreMesh(
    core_axis_name="core", subcore_axis_name="subcore")

_sc_params = pltpu.CompilerParams(
    needs_layout_passes=False, use_tc_tiling_on_sc=False)


def _dummy_dst(n, npad):
    return n + lax.rem(lax.iota(jnp.int32, LANES), jnp.int32(npad - n))


def _hist_sc(dst1d, n, npad, ept, tile_e):
    @functools.partial(
        pl.kernel,
        out_type=jax.ShapeDtypeStruct((NW, npad), jnp.float32),
        mesh=_vector_mesh,
        compiler_params=_sc_params,
        scratch_types=[
            pltpu.VMEM((tile_e,), jnp.int32),
            pltpu.VMEM((npad,), jnp.float32),
            pltpu.SemaphoreType.DMA,
        ],
    )
    def hist_k(dst_hbm, out_hbm, idx_v, hist_v, sem):
        c = lax.axis_index("core")
        s = lax.axis_index("subcore")
        w = c * NS + s
        cp = pltpu.async_copy(dst_hbm.at[pl.ds(w * ept, ept)],
                              idx_v.at[pl.ds(0, ept)], sem)
        dummy = _dummy_dst(n, npad)

        @pl.loop(ept, tile_e, step=LANES)
        def _(i):
            idx_v[pl.ds(i, LANES)] = dummy

        @pl.loop(0, npad, step=LANES)
        def _(i):
            hist_v[pl.ds(i, LANES)] = jnp.zeros((LANES,), jnp.float32)

        cp.wait()
        ones = jnp.ones((LANES,), jnp.float32)

        @pl.loop(0, tile_e, step=LANES)
        def _(e):
            idx = idx_v[pl.ds(e, LANES)]
            plsc.addupdate_scatter(hist_v, [idx], ones)

        pltpu.sync_copy(hist_v, out_hbm.at[w])

    return hist_k(dst1d)


def _prop_sc(g, src1d, dst1d, n, npad, ept, kr, rpt):
    hd = g.shape[1]
    tile_e = kr * CH
    NB = 8

    @functools.partial(
        pl.kernel,
        out_type=jax.ShapeDtypeStruct((NC, npad, hd), jnp.float32),
        mesh=_vector_mesh,
        compiler_params=_sc_params,
        scratch_types=[
            pltpu.VMEM((tile_e,), jnp.int32),
            pltpu.VMEM((tile_e,), jnp.int32),
            pltpu.VMEM((NB, CH, hd), jnp.float32),
            pltpu.VMEM((rpt, hd), jnp.float32),
            pltpu.VMEM_SHARED((npad, hd), jnp.float32),
            pltpu.SemaphoreType.DMA((NB,)),
            pltpu.SemaphoreType.DMA((NB,)),
        ],
    )
    def prop_k(g_hbm, src_hbm, dst_hbm, out_hbm,
               srcv, dstv, bufs, stage, accum, gsem, ssem):
        c = lax.axis_index("core")
        s = lax.axis_index("subcore")
        w = c * NS + s

        cp_s = pltpu.async_copy(src_hbm.at[pl.ds(w * ept, ept)],
                                srcv.at[pl.ds(0, ept)], gsem.at[0])
        cp_d = pltpu.async_copy(dst_hbm.at[pl.ds(w * ept, ept)],
                                dstv.at[pl.ds(0, ept)], gsem.at[1])
        dummy = _dummy_dst(n, npad)
        zeros16 = jnp.zeros((LANES,), jnp.int32)

        span = jnp.int32(npad - n)

        @pl.loop(ept, tile_e, step=LANES)
        def _(i):
            srcv[pl.ds(i, LANES)] = zeros16
            dstv[pl.ds(i, LANES)] = n + lax.rem(dummy - n + i, span)

        @pl.loop(0, rpt)
        def _(i):
            stage[i, :] = jnp.zeros((hd,), jnp.float32)

        pltpu.sync_copy(stage, accum.at[pl.ds(s * rpt, rpt)])
        cp_s.wait()
        cp_d.wait()
        plsc.subcore_barrier()

        def gather(m, k):
            pltpu.async_copy(g_hbm.at[srcv.at[pl.ds(m * CH, CH)]],
                             bufs.at[k], gsem.at[k])

        def gwait(k):
            pltpu.make_async_copy(g_hbm.at[srcv.at[pl.ds(0, CH)]],
                                  bufs.at[k], gsem.at[k]).wait()

        def scat(m, k):
            pltpu.async_copy(bufs.at[k], accum.at[dstv.at[pl.ds(m * CH, CH)]],
                             ssem.at[k], add=True)

        def swait(k):
            pltpu.make_async_copy(bufs.at[k], accum.at[dstv.at[pl.ds(0, CH)]],
                                  ssem.at[k]).wait()

        for k in range(NB // 2):
            gather(k, k)

        @pl.loop(0, kr, step=NB)
        def _(j):
            for k in range(NB):
                m = j + k

                @pl.when(m >= NB // 2)
                def _():
                    swait((k + NB // 2) % NB)

                @pl.when(m + NB // 2 < kr)
                def _():
                    gather(m + NB // 2, (k + NB // 2) % NB)

                gwait(k)
                scat(m, k)

        for k in range(NB // 2):
            swait((kr - NB // 2 + k) % NB)
        plsc.subcore_barrier()
        pltpu.sync_copy(accum.at[pl.ds(s * rpt, rpt)], stage)
        pltpu.sync_copy(stage, out_hbm.at[c, pl.ds(s * rpt, rpt)])

    return prop_k(g, src1d, dst1d)


def _tc_front(x, w1, hist):
    nw, npad = hist.shape
    n, f = x.shape
    hd = w1.shape[1]

    def body(x_ref, w_ref, hist_ref, dis_ref, g_ref):
        deg = jnp.sum(hist_ref[...], axis=0) + 1.0
        dis = lax.rsqrt(deg)
        disb = jnp.broadcast_to(dis[:, None], (npad, hd))
        dis_ref[...] = disb
        h = jnp.dot(x_ref[...], w_ref[...], preferred_element_type=jnp.float32)
        g_ref[...] = h * disb[:n, :]

    return pl.pallas_call(
        body,
        in_specs=[
            pl.BlockSpec((n, f), lambda: (0, 0)),
            pl.BlockSpec((f, hd), lambda: (0, 0)),
            pl.BlockSpec((nw, npad), lambda: (0, 0)),
        ],
        out_specs=[
            pl.BlockSpec((npad, hd), lambda: (0, 0)),
            pl.BlockSpec((n, hd), lambda: (0, 0)),
        ],
        out_shape=[
            jax.ShapeDtypeStruct((npad, hd), jnp.float32),
            jax.ShapeDtypeStruct((n, hd), jnp.float32),
        ],
    )(x, w1, hist)


def _tc2(p, g, dis, w2, b1):
    n, hd = g.shape
    npad = p.shape[1]

    def body(p_ref, g_ref, dis_ref, w_ref, b_ref, o_ref):
        s = p_ref[0, :n] + p_ref[1, :n] + g_ref[...]
        d = dis_ref[:n]
        a = jnp.maximum(d * s + b_ref[...], 0.0)
        h = jnp.dot(a, w_ref[...], preferred_element_type=jnp.float32)
        o_ref[...] = h * d

    return pl.pallas_call(
        body,
        in_specs=[
            pl.BlockSpec((NC, npad, hd), lambda: (0, 0, 0)),
            pl.BlockSpec((n, hd), lambda: (0, 0)),
            pl.BlockSpec((npad, hd), lambda: (0, 0)),
            pl.BlockSpec((hd, hd), lambda: (0, 0)),
            pl.BlockSpec((1, hd), lambda: (0, 0)),
        ],
        out_specs=pl.BlockSpec((n, hd), lambda: (0, 0)),
        out_shape=jax.ShapeDtypeStruct((n, hd), jnp.float32),
    )(p, g, dis, w2, b1)


def _tc3(q, g2, dis, b2, wl, bl):
    n, hd = g2.shape
    npad = q.shape[1]
    co = wl.shape[1]

    def body(q_ref, g_ref, dis_ref, b_ref, w_ref, bl_ref, o_ref):
        s = q_ref[0, :n] + q_ref[1, :n] + g_ref[...]
        a = jnp.maximum(dis_ref[:n] * s + b_ref[...], 0.0)
        logits = jnp.dot(a, w_ref[...], preferred_element_type=jnp.float32)
        logits = logits + bl_ref[...]
        m = jnp.max(logits, axis=1, keepdims=True)
        lse = m + jnp.log(jnp.sum(jnp.exp(logits - m), axis=1, keepdims=True))
        o_ref[...] = logits - lse

    return pl.pallas_call(
        body,
        in_specs=[
            pl.BlockSpec((NC, npad, hd), lambda: (0, 0, 0)),
            pl.BlockSpec((n, hd), lambda: (0, 0)),
            pl.BlockSpec((npad, hd), lambda: (0, 0)),
            pl.BlockSpec((1, hd), lambda: (0, 0)),
            pl.BlockSpec((hd, co), lambda: (0, 0)),
            pl.BlockSpec((1, co), lambda: (0, 0)),
        ],
        out_specs=pl.BlockSpec((n, co), lambda: (0, 0)),
        out_shape=jax.ShapeDtypeStruct((n, co), jnp.float32),
    )(q, g2, dis, b2, wl, bl)


def kernel(x, edge_index, W1, b1, W2, b2, Wl, bl):
    n = x.shape[0]
    e = edge_index.shape[1]

    e1 = -(-e // (NW * LANES)) * (NW * LANES)
    src = edge_index[0]
    dst = edge_index[1]
    if e1 != e:
        src = jnp.concatenate([src, jnp.zeros((e1 - e,), jnp.int32)])
        dst = jnp.concatenate([dst, jnp.full((e1 - e,), n, jnp.int32)])
    ept = e1 // NW
    tile_e = -(-ept // (8 * CH)) * (8 * CH)
    kr = tile_e // CH
    npad = -(-(n + 1) // (8 * NS)) * (8 * NS)
    rpt = npad // NS

    hist = _hist_sc(dst, n, npad, ept, tile_e)
    dis, g1 = _tc_front(x, W1, hist)
    p = _prop_sc(g1, src, dst, n, npad, ept, kr, rpt)
    g2 = _tc2(p, g1, dis, W2, b1.reshape(1, -1))
    q = _prop_sc(g2, src, dst, n, npad, ept, kr, rpt)
    return _tc3(q, g2, dis, b2.reshape(1, -1), Wl, bl.reshape(1, -1))


# speedup vs baseline: 1.3025x; 1.0659x over previous
import dataclasses
import functools

import jax
import jax.numpy as jnp
from jax import lax
from jax.experimental import pallas as pl
from jax.experimental.pallas import tpu as pltpu
from jax.experimental.pallas import tpu_sc as plsc

NC = 2
NS = 16
NW = NC * NS
CH = 128
LANES = 16

_vector_mesh = plsc.VectorSubcoreMesh(
    core_axis_name="core", subcore_axis_name="subcore")

_sc_params = pltpu.CompilerParams(
    needs_layout_passes=False, use_tc_tiling_on_sc=False)


def _tc_edges(edge_index):
    e = edge_index.shape[1]

    def body(ei_ref, s_ref, d_ref):
        s_ref[...] = ei_ref[0]
        d_ref[...] = ei_ref[1]

    return pl.pallas_call(
        body,
        in_specs=[pl.BlockSpec((2, e), lambda: (0, 0))],
        out_specs=[
            pl.BlockSpec((e,), lambda: (0,)),
            pl.BlockSpec((e,), lambda: (0,)),
        ],
        out_shape=[
            jax.ShapeDtypeStruct((e,), jnp.int32),
            jax.ShapeDtypeStruct((e,), jnp.int32),
        ],
    )(edge_index)


def _dummy_dst(n, npad):
    return n + lax.rem(lax.iota(jnp.int32, LANES), jnp.int32(npad - n))


def _hist_sc(dst1d, n, npad, ept, tile_e):
    @functools.partial(
        pl.kernel,
        out_type=jax.ShapeDtypeStruct((NW, npad), jnp.float32),
        mesh=_vector_mesh,
        compiler_params=_sc_params,
        scratch_types=[
            pltpu.VMEM((tile_e,), jnp.int32),
            pltpu.VMEM((npad,), jnp.float32),
            pltpu.SemaphoreType.DMA,
        ],
    )
    def hist_k(dst_hbm, out_hbm, idx_v, hist_v, sem):
        c = lax.axis_index("core")
        s = lax.axis_index("subcore")
        w = c * NS + s
        cp = pltpu.async_copy(dst_hbm.at[pl.ds(w * ept, ept)],
                              idx_v.at[pl.ds(0, ept)], sem)
        dummy = _dummy_dst(n, npad)

        @pl.loop(ept, tile_e, step=LANES)
        def _(i):
            idx_v[pl.ds(i, LANES)] = dummy

        @pl.loop(0, npad, step=LANES)
        def _(i):
            hist_v[pl.ds(i, LANES)] = jnp.zeros((LANES,), jnp.float32)

        cp.wait()
        ones = jnp.ones((LANES,), jnp.float32)

        @pl.loop(0, tile_e, step=LANES)
        def _(e):
            idx = idx_v[pl.ds(e, LANES)]
            plsc.addupdate_scatter(hist_v, [idx], ones)

        pltpu.sync_copy(hist_v, out_hbm.at[w])

    return hist_k(dst1d)


def _prop_sc(g, src1d, dst1d, n, npad, ept, kr, rpt):
    hd = g.shape[1]
    tile_e = kr * CH
    NB = 8

    @functools.partial(
        pl.kernel,
        out_type=jax.ShapeDtypeStruct((NC, npad, hd), jnp.float32),
        mesh=_vector_mesh,
        compiler_params=_sc_params,
        scratch_types=[
            pltpu.VMEM((tile_e,), jnp.int32),
            pltpu.VMEM((tile_e,), jnp.int32),
            pltpu.VMEM((NB, CH, hd), jnp.float32),
            pltpu.VMEM((rpt, hd), jnp.float32),
            pltpu.VMEM_SHARED((npad, hd), jnp.float32),
            pltpu.SemaphoreType.DMA((NB,)),
            pltpu.SemaphoreType.DMA((NB,)),
        ],
    )
    def prop_k(g_hbm, src_hbm, dst_hbm, out_hbm,
               srcv, dstv, bufs, stage, accum, gsem, ssem):
        c = lax.axis_index("core")
        s = lax.axis_index("subcore")
        w = c * NS + s

        cp_s = pltpu.async_copy(src_hbm.at[pl.ds(w * ept, ept)],
                                srcv.at[pl.ds(0, ept)], gsem.at[0])
        cp_d = pltpu.async_copy(dst_hbm.at[pl.ds(w * ept, ept)],
                                dstv.at[pl.ds(0, ept)], gsem.at[1])
        dummy = _dummy_dst(n, npad)
        zeros16 = jnp.zeros((LANES,), jnp.int32)

        span = jnp.int32(npad - n)

        @pl.loop(ept, tile_e, step=LANES)
        def _(i):
            srcv[pl.ds(i, LANES)] = zeros16
            dstv[pl.ds(i, LANES)] = n + lax.rem(dummy - n + i, span)

        @pl.loop(0, rpt)
        def _(i):
            stage[i, :] = jnp.zeros((hd,), jnp.float32)

        pltpu.sync_copy(stage, accum.at[pl.ds(s * rpt, rpt)])
        cp_s.wait()
        cp_d.wait()
        plsc.subcore_barrier()

        def gather(m, k):
            pltpu.async_copy(g_hbm.at[srcv.at[pl.ds(m * CH, CH)]],
                             bufs.at[k], gsem.at[k])

        def gwait(k):
            pltpu.make_async_copy(g_hbm.at[srcv.at[pl.ds(0, CH)]],
                                  bufs.at[k], gsem.at[k]).wait()

        def scat(m, k):
            pltpu.async_copy(bufs.at[k], accum.at[dstv.at[pl.ds(m * CH, CH)]],
                             ssem.at[k], add=True)

        def swait(k):
            pltpu.make_async_copy(bufs.at[k], accum.at[dstv.at[pl.ds(0, CH)]],
                                  ssem.at[k]).wait()

        for k in range(NB // 2):
            gather(k, k)

        @pl.loop(0, kr, step=NB)
        def _(j):
            for k in range(NB):
                m = j + k

                @pl.when(m >= NB // 2)
                def _():
                    swait((k + NB // 2) % NB)

                @pl.when(m + NB // 2 < kr)
                def _():
                    gather(m + NB // 2, (k + NB // 2) % NB)

                gwait(k)
                scat(m, k)

        for k in range(NB // 2):
            swait((kr - NB // 2 + k) % NB)
        plsc.subcore_barrier()
        pltpu.sync_copy(accum.at[pl.ds(s * rpt, rpt)], stage)
        pltpu.sync_copy(stage, out_hbm.at[c, pl.ds(s * rpt, rpt)])

    return prop_k(g, src1d, dst1d)


def _tc_front(x, w1, hist):
    nw, npad = hist.shape
    n, f = x.shape
    hd = w1.shape[1]

    def body(x_ref, w_ref, hist_ref, dis_ref, g_ref):
        deg = jnp.sum(hist_ref[...], axis=0) + 1.0
        dis = lax.rsqrt(deg)
        disb = jnp.broadcast_to(dis[:, None], (npad, hd))
        dis_ref[...] = disb
        h = jnp.dot(x_ref[...].astype(jnp.bfloat16),
                    w_ref[...].astype(jnp.bfloat16),
                    preferred_element_type=jnp.float32)
        g_ref[...] = h * disb[:n, :]

    return pl.pallas_call(
        body,
        in_specs=[
            pl.BlockSpec((n, f), lambda: (0, 0)),
            pl.BlockSpec((f, hd), lambda: (0, 0)),
            pl.BlockSpec((nw, npad), lambda: (0, 0)),
        ],
        out_specs=[
            pl.BlockSpec((npad, hd), lambda: (0, 0)),
            pl.BlockSpec((n, hd), lambda: (0, 0)),
        ],
        out_shape=[
            jax.ShapeDtypeStruct((npad, hd), jnp.float32),
            jax.ShapeDtypeStruct((n, hd), jnp.float32),
        ],
    )(x, w1, hist)


def _tc2(p, g, dis, w2, b1):
    n, hd = g.shape
    npad = p.shape[1]

    def body(p_ref, g_ref, dis_ref, w_ref, b_ref, o_ref):
        s = p_ref[0, :n] + p_ref[1, :n] + g_ref[...]
        d = dis_ref[:n]
        a = jnp.maximum(d * s + b_ref[...], 0.0)
        h = jnp.dot(a.astype(jnp.bfloat16), w_ref[...].astype(jnp.bfloat16),
                    preferred_element_type=jnp.float32)
        o_ref[...] = h * d

    return pl.pallas_call(
        body,
        in_specs=[
            pl.BlockSpec((NC, npad, hd), lambda: (0, 0, 0)),
            pl.BlockSpec((n, hd), lambda: (0, 0)),
            pl.BlockSpec((npad, hd), lambda: (0, 0)),
            pl.BlockSpec((hd, hd), lambda: (0, 0)),
            pl.BlockSpec((1, hd), lambda: (0, 0)),
        ],
        out_specs=pl.BlockSpec((n, hd), lambda: (0, 0)),
        out_shape=jax.ShapeDtypeStruct((n, hd), jnp.float32),
    )(p, g, dis, w2, b1)


def _tc3(q, g2, dis, b2, wl, bl):
    n, hd = g2.shape
    npad = q.shape[1]
    co = wl.shape[1]

    def body(q_ref, g_ref, dis_ref, b_ref, w_ref, bl_ref, o_ref):
        s = q_ref[0, :n] + q_ref[1, :n] + g_ref[...]
        a = jnp.maximum(dis_ref[:n] * s + b_ref[...], 0.0)
        logits = jnp.dot(a.astype(jnp.bfloat16), w_ref[...].astype(jnp.bfloat16),
                         preferred_element_type=jnp.float32)
        logits = logits + bl_ref[...]
        m = jnp.max(logits, axis=1, keepdims=True)
        lse = m + jnp.log(jnp.sum(jnp.exp(logits - m), axis=1, keepdims=True))
        o_ref[...] = logits - lse

    return pl.pallas_call(
        body,
        in_specs=[
            pl.BlockSpec((NC, npad, hd), lambda: (0, 0, 0)),
            pl.BlockSpec((n, hd), lambda: (0, 0)),
            pl.BlockSpec((npad, hd), lambda: (0, 0)),
            pl.BlockSpec((1, hd), lambda: (0, 0)),
            pl.BlockSpec((hd, co), lambda: (0, 0)),
            pl.BlockSpec((1, co), lambda: (0, 0)),
        ],
        out_specs=pl.BlockSpec((n, co), lambda: (0, 0)),
        out_shape=jax.ShapeDtypeStruct((n, co), jnp.float32),
    )(q, g2, dis, b2, wl, bl)


def kernel(x, edge_index, W1, b1, W2, b2, Wl, bl):
    n = x.shape[0]
    e = edge_index.shape[1]

    e1 = -(-e // (NW * LANES)) * (NW * LANES)
    src, dst = _tc_edges(edge_index)
    if e1 != e:
        src = jnp.concatenate([src, jnp.zeros((e1 - e,), jnp.int32)])
        dst = jnp.concatenate([dst, jnp.full((e1 - e,), n, jnp.int32)])
    ept = e1 // NW
    tile_e = -(-ept // (8 * CH)) * (8 * CH)
    kr = tile_e // CH
    npad = -(-(n + 1) // (8 * NS)) * (8 * NS)
    rpt = npad // NS

    hist = _hist_sc(dst, n, npad, ept, tile_e)
    dis, g1 = _tc_front(x, W1, hist)
    p = _prop_sc(g1, src, dst, n, npad, ept, kr, rpt)
    g2 = _tc2(p, g1, dis, W2, b1.reshape(1, -1))
    q = _prop_sc(g2, src, dst, n, npad, ept, kr, rpt)
    return _tc3(q, g2, dis, b2.reshape(1, -1), Wl, bl.reshape(1, -1))


# speedup vs baseline: 1.3031x; 1.0004x over previous
import functools

import jax
import jax.numpy as jnp
from jax import lax
from jax.experimental import pallas as pl
from jax.experimental.pallas import tpu as pltpu
from jax.experimental.pallas import tpu_sc as plsc

NC = 2
NS = 16
NW = NC * NS
CH = 128
LANES = 16

def _vmesh():
    return plsc.VectorSubcoreMesh(
        core_axis_name="core", subcore_axis_name="subcore")


_sc_params = pltpu.CompilerParams(
    needs_layout_passes=False, use_tc_tiling_on_sc=False)


def _tc_edges(edge_index):
    e = edge_index.shape[1]

    def body(ei_ref, s_ref, d_ref):
        s_ref[...] = ei_ref[0]
        d_ref[...] = ei_ref[1]

    return pl.pallas_call(
        body,
        in_specs=[pl.BlockSpec((2, e), lambda: (0, 0))],
        out_specs=[
            pl.BlockSpec((e,), lambda: (0,)),
            pl.BlockSpec((e,), lambda: (0,)),
        ],
        out_shape=[
            jax.ShapeDtypeStruct((e,), jnp.int32),
            jax.ShapeDtypeStruct((e,), jnp.int32),
        ],
    )(edge_index)


def _dummy_dst(n, npad):
    return n + lax.rem(lax.iota(jnp.int32, LANES), jnp.int32(npad - n))


def _hist_sc(dst1d, n, npad, ept, tile_e):
    @functools.partial(
        pl.kernel,
        out_type=jax.ShapeDtypeStruct((NW, npad), jnp.float32),
        mesh=_vmesh(),
        compiler_params=_sc_params,
        scratch_types=[
            pltpu.VMEM((tile_e,), jnp.int32),
            pltpu.VMEM((npad,), jnp.float32),
            pltpu.SemaphoreType.DMA,
        ],
    )
    def hist_k(dst_hbm, out_hbm, idx_v, hist_v, sem):
        c = lax.axis_index("core")
        s = lax.axis_index("subcore")
        w = c * NS + s
        cp = pltpu.async_copy(dst_hbm.at[pl.ds(w * ept, ept)],
                              idx_v.at[pl.ds(0, ept)], sem)
        dummy = _dummy_dst(n, npad)

        @pl.loop(ept, tile_e, step=LANES)
        def _(i):
            idx_v[pl.ds(i, LANES)] = dummy

        @pl.loop(0, npad, step=LANES)
        def _(i):
            hist_v[pl.ds(i, LANES)] = jnp.zeros((LANES,), jnp.float32)

        cp.wait()
        ones = jnp.ones((LANES,), jnp.float32)

        @pl.loop(0, tile_e, step=LANES)
        def _(e):
            idx = idx_v[pl.ds(e, LANES)]
            plsc.addupdate_scatter(hist_v, [idx], ones)

        pltpu.sync_copy(hist_v, out_hbm.at[w])

    return hist_k(dst1d)


def _prop_sc(g, src1d, dst1d, n, npad, ept, kr, rpt):
    hd = g.shape[1]
    tile_e = kr * CH
    NB = 8

    @functools.partial(
        pl.kernel,
        out_type=jax.ShapeDtypeStruct((NC, npad, hd), jnp.float32),
        mesh=_vmesh(),
        compiler_params=_sc_params,
        scratch_types=[
            pltpu.VMEM((tile_e,), jnp.int32),
            pltpu.VMEM((tile_e,), jnp.int32),
            pltpu.VMEM((NB, CH, hd), jnp.float32),
            pltpu.VMEM((rpt, hd), jnp.float32),
            pltpu.VMEM_SHARED((npad, hd), jnp.float32),
            pltpu.SemaphoreType.DMA((NB,)),
            pltpu.SemaphoreType.DMA((NB,)),
        ],
    )
    def prop_k(g_hbm, src_hbm, dst_hbm, out_hbm,
               srcv, dstv, bufs, stage, accum, gsem, ssem):
        c = lax.axis_index("core")
        s = lax.axis_index("subcore")
        w = c * NS + s

        cp_s = pltpu.async_copy(src_hbm.at[pl.ds(w * ept, ept)],
                                srcv.at[pl.ds(0, ept)], gsem.at[0])
        cp_d = pltpu.async_copy(dst_hbm.at[pl.ds(w * ept, ept)],
                                dstv.at[pl.ds(0, ept)], gsem.at[1])
        dummy = _dummy_dst(n, npad)
        zeros16 = jnp.zeros((LANES,), jnp.int32)

        span = jnp.int32(npad - n)

        @pl.loop(ept, tile_e, step=LANES)
        def _(i):
            srcv[pl.ds(i, LANES)] = zeros16
            dstv[pl.ds(i, LANES)] = n + lax.rem(dummy - n + i, span)

        @pl.loop(0, rpt)
        def _(i):
            stage[i, :] = jnp.zeros((hd,), jnp.float32)

        pltpu.sync_copy(stage, accum.at[pl.ds(s * rpt, rpt)])
        cp_s.wait()
        cp_d.wait()
        plsc.subcore_barrier()

        def gather(m, k):
            pltpu.async_copy(g_hbm.at[srcv.at[pl.ds(m * CH, CH)]],
                             bufs.at[k], gsem.at[k])

        def gwait(k):
            pltpu.make_async_copy(g_hbm.at[srcv.at[pl.ds(0, CH)]],
                                  bufs.at[k], gsem.at[k]).wait()

        def scat(m, k):
            pltpu.async_copy(bufs.at[k], accum.at[dstv.at[pl.ds(m * CH, CH)]],
                             ssem.at[k], add=True)

        def swait(k):
            pltpu.make_async_copy(bufs.at[k], accum.at[dstv.at[pl.ds(0, CH)]],
                                  ssem.at[k]).wait()

        for k in range(NB // 2):
            gather(k, k)

        @pl.loop(0, kr, step=NB)
        def _(j):
            for k in range(NB):
                m = j + k

                @pl.when(m >= NB // 2)
                def _():
                    swait((k + NB // 2) % NB)

                @pl.when(m + NB // 2 < kr)
                def _():
                    gather(m + NB // 2, (k + NB // 2) % NB)

                gwait(k)
                scat(m, k)

        for k in range(NB // 2):
            swait((kr - NB // 2 + k) % NB)
        plsc.subcore_barrier()
        pltpu.sync_copy(accum.at[pl.ds(s * rpt, rpt)], stage)
        pltpu.sync_copy(stage, out_hbm.at[c, pl.ds(s * rpt, rpt)])

    return prop_k(g, src1d, dst1d)


def _tc_front(x, w1, hist):
    nw, npad = hist.shape
    n, f = x.shape
    hd = w1.shape[1]

    def body(x_ref, w_ref, hist_ref, dis_ref, g_ref):
        deg = jnp.sum(hist_ref[...], axis=0) + 1.0
        dis = lax.rsqrt(deg)
        disb = jnp.broadcast_to(dis[:, None], (npad, hd))
        dis_ref[...] = disb
        h = jnp.dot(x_ref[...].astype(jnp.bfloat16),
                    w_ref[...].astype(jnp.bfloat16),
                    preferred_element_type=jnp.float32)
        g_ref[...] = h * disb[:n, :]

    return pl.pallas_call(
        body,
        in_specs=[
            pl.BlockSpec((n, f), lambda: (0, 0)),
            pl.BlockSpec((f, hd), lambda: (0, 0)),
            pl.BlockSpec((nw, npad), lambda: (0, 0)),
        ],
        out_specs=[
            pl.BlockSpec((npad, hd), lambda: (0, 0)),
            pl.BlockSpec((n, hd), lambda: (0, 0)),
        ],
        out_shape=[
            jax.ShapeDtypeStruct((npad, hd), jnp.float32),
            jax.ShapeDtypeStruct((n, hd), jnp.float32),
        ],
    )(x, w1, hist)


def _tc2(p, g, dis, w2, b1):
    n, hd = g.shape
    npad = p.shape[1]

    def body(p_ref, g_ref, dis_ref, w_ref, b_ref, o_ref):
        s = p_ref[0, :n] + p_ref[1, :n] + g_ref[...]
        d = dis_ref[:n]
        a = jnp.maximum(d * s + b_ref[...], 0.0)
        h = jnp.dot(a.astype(jnp.bfloat16), w_ref[...].astype(jnp.bfloat16),
                    preferred_element_type=jnp.float32)
        o_ref[...] = h * d

    return pl.pallas_call(
        body,
        in_specs=[
            pl.BlockSpec((NC, npad, hd), lambda: (0, 0, 0)),
            pl.BlockSpec((n, hd), lambda: (0, 0)),
            pl.BlockSpec((npad, hd), lambda: (0, 0)),
            pl.BlockSpec((hd, hd), lambda: (0, 0)),
            pl.BlockSpec((1, hd), lambda: (0, 0)),
        ],
        out_specs=pl.BlockSpec((n, hd), lambda: (0, 0)),
        out_shape=jax.ShapeDtypeStruct((n, hd), jnp.float32),
    )(p, g, dis, w2, b1)


def _tc3(q, g2, dis, b2, wl, bl):
    n, hd = g2.shape
    npad = q.shape[1]
    co = wl.shape[1]

    def body(q_ref, g_ref, dis_ref, b_ref, w_ref, bl_ref, o_ref):
        s = q_ref[0, :n] + q_ref[1, :n] + g_ref[...]
        a = jnp.maximum(dis_ref[:n] * s + b_ref[...], 0.0)
        logits = jnp.dot(a.astype(jnp.bfloat16), w_ref[...].astype(jnp.bfloat16),
                         preferred_element_type=jnp.float32)
        logits = logits + bl_ref[...]
        m = jnp.max(logits, axis=1, keepdims=True)
        lse = m + jnp.log(jnp.sum(jnp.exp(logits - m), axis=1, keepdims=True))
        o_ref[...] = logits - lse

    return pl.pallas_call(
        body,
        in_specs=[
            pl.BlockSpec((NC, npad, hd), lambda: (0, 0, 0)),
            pl.BlockSpec((n, hd), lambda: (0, 0)),
            pl.BlockSpec((npad, hd), lambda: (0, 0)),
            pl.BlockSpec((1, hd), lambda: (0, 0)),
            pl.BlockSpec((hd, co), lambda: (0, 0)),
            pl.BlockSpec((1, co), lambda: (0, 0)),
        ],
        out_specs=pl.BlockSpec((n, co), lambda: (0, 0)),
        out_shape=jax.ShapeDtypeStruct((n, co), jnp.float32),
    )(q, g2, dis, b2, wl, bl)


def kernel(x, edge_index, W1, b1, W2, b2, Wl, bl):
    n = x.shape[0]
    e = edge_index.shape[1]

    e1 = -(-e // (NW * LANES)) * (NW * LANES)
    src, dst = _tc_edges(edge_index)
    if e1 != e:
        src = jnp.concatenate([src, jnp.zeros((e1 - e,), jnp.int32)])
        dst = jnp.concatenate([dst, jnp.full((e1 - e,), n, jnp.int32)])
    ept = e1 // NW
    tile_e = -(-ept // (8 * CH)) * (8 * CH)
    kr = tile_e // CH
    npad = -(-(n + 1) // (8 * NS)) * (8 * NS)
    rpt = npad // NS

    hist = _hist_sc(dst, n, npad, ept, tile_e)
    dis, g1 = _tc_front(x, W1, hist)
    p = _prop_sc(g1, src, dst, n, npad, ept, kr, rpt)
    g2 = _tc2(p, g1, dis, W2, b1.reshape(1, -1))
    q = _prop_sc(g2, src, dst, n, npad, ept, kr, rpt)
    return _tc3(q, g2, dis, b2.reshape(1, -1), Wl, bl.reshape(1, -1))
